# baseline (speedup 1.0000x reference)
import functools

import jax
import jax.numpy as jnp
from jax import lax
from jax.experimental import pallas as pl
from jax.experimental.pallas import tpu as pltpu
from jax.experimental.pallas import tpu_sc as plsc

_NC = 2
_NS = 16
_LANES = 16
_CHUNK = 128


def _sc_extract(L, B, NK, K, scores, mnn_batch, mnn_a, mnn_b):
    P = L * B
    NW = NK // _CHUNK
    assert P + 2 <= _NC * _NS

    mesh = plsc.VectorSubcoreMesh(core_axis_name="c", subcore_axis_name="s")

    @functools.partial(
        pl.kernel,
        out_type=(
            jax.ShapeDtypeStruct((L * K, _CHUNK), jnp.float32),
            jax.ShapeDtypeStruct((B * NK,), jnp.float32),
            jax.ShapeDtypeStruct((B * NK,), jnp.float32),
        ),
        mesh=mesh,
        scratch_types=[
            pltpu.VMEM((K,), jnp.int32),
            pltpu.VMEM((K,), jnp.int32),
            pltpu.VMEM((K,), jnp.int32),
            pltpu.VMEM((K + _LANES,), jnp.int32),
            pltpu.VMEM((K + _LANES,), jnp.int32),
            pltpu.VMEM((_LANES, _CHUNK), jnp.float32),
            pltpu.VMEM((NK,), jnp.float32),
            pltpu.VMEM((K // _CHUNK, _CHUNK), jnp.int32),
            pltpu.VMEM((_CHUNK,), jnp.float32),
            pltpu.SemaphoreType.DMA,
        ],
        compiler_params=pltpu.CompilerParams(needs_layout_passes=False),
    )
    def sc_kernel(scores_hbm, mb_hbm, ma_hbm, mbb_hbm,
                  win_hbm, mA_hbm, mB_hbm,
                  bi_v, ai_v, ci_v, klist_v, klist2_v, rows_v,
                  zbuf_v, idx2_v, ones_v, sem):
        wid = lax.axis_index("s") * _NC + lax.axis_index("c")
        lane = lax.iota(jnp.int32, _LANES)

        @pl.when(wid < P)
        def _():
            l = wid // B
            b = wid % B
            pltpu.sync_copy(mb_hbm, bi_v)
            pltpu.sync_copy(ma_hbm, ai_v)
            pltpu.sync_copy(mbb_hbm, ci_v)

            zero16 = jnp.zeros((_LANES,), jnp.int32)

            def zklist(t, _):
                klist_v[pl.ds(t * _LANES, _LANES)] = zero16
                klist2_v[pl.ds(t * _LANES, _LANES)] = zero16
                return 0
            lax.fori_loop(0, (K + _LANES) // _LANES, zklist, 0, unroll=8)

            def comp(t, off):
                o = t * _LANES
                kidx = o + lane
                m = bi_v[pl.ds(o, _LANES)] == b
                pos = plsc.cumsum(m.astype(jnp.int32)) - 1 + off
                plsc.store_scatter(klist_v, [pos], kidx, mask=m)
                return off + jnp.sum(m.astype(jnp.int32))
            cnt = lax.fori_loop(0, K // _LANES, comp, 0, unroll=8)
            nch = (cnt + _LANES - 1) // _LANES

            def wbody(w, _):
                def comp2(t, off):
                    o = t * _LANES
                    kc = klist_v[pl.ds(o, _LANES)]
                    valid = (o + lane) < cnt
                    col = plsc.load_gather(ci_v, [kc])
                    m = jnp.logical_and(valid, (col // _CHUNK) == w)
                    pos = plsc.cumsum(m.astype(jnp.int32)) - 1 + off
                    plsc.store_scatter(klist2_v, [pos], kc, mask=m)
                    return off + jnp.sum(m.astype(jnp.int32))
                cntw = lax.fori_loop(0, nch, comp2, 0)

                def rowloop(c, _):
                    o = c * _LANES
                    kc = klist2_v[pl.ds(o, _LANES)]
                    valid = (o + lane) < cntw
                    row_i = plsc.load_gather(ai_v, [kc])
                    pltpu.async_copy(
                        scores_hbm.at[l, b].at[row_i,
                                               pl.ds(w * _CHUNK, _CHUNK)],
                        rows_v, sem).wait()
                    out_idx = jnp.where(valid, l * K + kc, -1)
                    pltpu.async_copy(
                        rows_v,
                        win_hbm.at[plsc.Indices(out_idx, ignored_value=-1)],
                        sem).wait()
                    return 0
                lax.fori_loop(0, (cntw + _LANES - 1) // _LANES, rowloop, 0)
                return 0
            lax.fori_loop(0, NW, wbody, 0)

        def build_mask(key_ref, out_hbm):
            def zero(t, _):
                zbuf_v[pl.ds(t * _LANES, _LANES)] = jnp.zeros(
                    (_LANES,), jnp.float32)
                return 0
            lax.fori_loop(0, NK // _LANES, zero, 0, unroll=8)
            for q in range((B * NK) // NK):
                pltpu.sync_copy(zbuf_v, out_hbm.at[pl.ds(q * NK, NK)])
            for j in range(_CHUNK // _LANES):
                ones_v[pl.ds(j * _LANES, _LANES)] = jnp.ones(
                    (_LANES,), jnp.float32)
            pltpu.sync_copy(mb_hbm, bi_v)
            pltpu.sync_copy(key_ref, ai_v)
            for ci in range(K // _CHUNK):
                for j in range(_CHUNK // _LANES):
                    o = ci * _CHUNK + j * _LANES
                    key = (bi_v[pl.ds(o, _LANES)] * NK
                           + ai_v[pl.ds(o, _LANES)])
                    idx2_v[ci, pl.ds(j * _LANES, _LANES)] = key
            copies = []
            for ci in range(K // _CHUNK):
                copies.append(
                    pltpu.async_copy(ones_v, out_hbm.at[idx2_v.at[ci]], sem))
            for cp in copies:
                cp.wait()

        @pl.when(wid == P)
        def _():
            build_mask(ma_hbm, mA_hbm)

        @pl.when(wid == P + 1)
        def _():
            build_mask(mbb_hbm, mB_hbm)

    return sc_kernel(scores, mnn_batch, mnn_a, mnn_b)


def _tc_reduce(L, B, NK, K, scores, win, mbbcol, mA, mB):
    P = L * B
    KB = K // B
    NWIN = (NK + 1 + _CHUNK - 1) // _CHUNK - 1

    def body(colb_ref, rowb_ref, win_ref, mbb_ref, mA_ref, mB_ref, out_ref):
        p = pl.program_id(0)
        b = p % B

        def sp(v):
            return jnp.maximum(v, 0.0) + jnp.log1p(jnp.exp(-jnp.abs(v)))

        c1 = colb_ref[0, 0, :, pl.ds(0, 1)]
        r1 = rowb_ref[0, 0, pl.ds(0, 1), :]
        w = win_ref[...]
        mbb = mbb_ref[pl.ds(b * KB, KB), :]
        ma1 = mA_ref[pl.ds(b, 1), :]
        mb1 = mB_ref[pl.ds(b, 1), :]

        oh = (mbb == lax.broadcasted_iota(
            jnp.int32, (1, _CHUNK), 1)).astype(jnp.float32)
        maT = jnp.transpose(ma1)

        partial = ((jnp.sum(sp(c1)) - jnp.sum(c1 * maT)
                    + jnp.sum(sp(r1)) - jnp.sum(r1 * mb1)) / (L * B * NK)
                   - jnp.sum(w * oh) / (L * K))

        @pl.when(p == 0)
        def _():
            out_ref[...] = jnp.zeros((1, 1), jnp.float32)
        out_ref[...] += jnp.broadcast_to(partial, (1, 1))

    out = pl.pallas_call(
        body,
        grid=(P,),
        in_specs=[
            pl.BlockSpec((1, 1, NK, _CHUNK),
                         lambda p: (p // B, p % B, 0, NWIN)),
            pl.BlockSpec((1, 1, 8, NK),
                         lambda p: (p // B, p % B, NK // 8, 0)),
            pl.BlockSpec((KB, _CHUNK), lambda p: (p, 0)),
            pl.BlockSpec((K, 1), lambda p: (0, 0)),
            pl.BlockSpec((B, NK), lambda p: (0, 0)),
            pl.BlockSpec((B, NK), lambda p: (0, 0)),
        ],
        out_specs=pl.BlockSpec((1, 1), lambda p: (0, 0)),
        out_shape=jax.ShapeDtypeStruct((1, 1), jnp.float32),
    )(scores, scores, win, mbbcol, mA, mB)
    return out.reshape(())


def kernel(scores, mnn_batch, mnn_a, mnn_b):
    L, B, Mp1, Np1 = scores.shape
    NK = Mp1 - 1
    K = mnn_batch.shape[0]
    P = L * B
    assert Mp1 == Np1 and NK % _CHUNK == 0 and K % _CHUNK == 0
    assert (B * NK) % _LANES == 0 and K % B == 0

    win_f, mA_f, mB_f = _sc_extract(
        L, B, NK, K, scores,
        mnn_batch.astype(jnp.int32), mnn_a.astype(jnp.int32),
        mnn_b.astype(jnp.int32))

    return _tc_reduce(
        L, B, NK, K, scores,
        win_f,
        (mnn_b.astype(jnp.int32) % _CHUNK).reshape(K, 1),
        mA_f.reshape(B, NK),
        mB_f.reshape(B, NK))
